# baseline (speedup 1.0000x reference)
import functools

import jax
import jax.numpy as jnp
from jax import lax
from jax.experimental import pallas as pl
from jax.experimental.pallas import tpu as pltpu
from jax.experimental.pallas import tpu_sc as plsc

NUM_CORES = 2
NUM_SUBCORES = 16
NUM_WORKERS = NUM_CORES * NUM_SUBCORES
LANES = 16
BW = 128
NBUF = 6


@functools.partial(jax.jit, static_argnums=(3, 4))
def _embed(ids_t, token_table, pos_s, S, D):
    n_chunks = ids_t.shape[0]
    nblk = n_chunks // S
    n_rows = n_chunks * BW
    chunks_per_w = n_chunks // NUM_WORKERS
    n_full = chunks_per_w // NBUF
    n_tail = chunks_per_w - n_full * NBUF
    vregs_per_row = D // LANES
    stage_rows = chunks_per_w + 6

    mesh = plsc.VectorSubcoreMesh(
        core_axis_name="c", subcore_axis_name="s")

    @functools.partial(
        pl.kernel,
        mesh=mesh,
        out_type=jax.ShapeDtypeStruct((n_rows, D), jnp.float32),
        scratch_types=[
            pltpu.VMEM((stage_rows, BW), jnp.int32),
            pltpu.VMEM((16, D), jnp.float32),
            pltpu.VMEM((NBUF, BW, D), jnp.float32),
            pltpu.VMEM((NBUF, BW), jnp.int32),
            pltpu.VMEM((BW,), jnp.int32),
            pltpu.SemaphoreType.DMA,
            pltpu.SemaphoreType.DMA,
            pltpu.SemaphoreType.DMA,
            pltpu.SemaphoreType.DMA,
            pltpu.SemaphoreType.DMA,
            pltpu.SemaphoreType.DMA,
            pltpu.SemaphoreType.DMA,
            pltpu.SemaphoreType.DMA,
            pltpu.SemaphoreType.DMA,
            pltpu.SemaphoreType.DMA,
            pltpu.SemaphoreType.DMA,
            pltpu.SemaphoreType.DMA,
            pltpu.SemaphoreType.DMA,
            pltpu.SemaphoreType.DMA,
        ],
    )
    def body(ids_hbm, ttab_hbm, ptab_hbm, out_hbm, idx_v, pos_v, bufs,
             oidx_v, iota_v, g0, g1, g2, g3, g4, g5, s0, s1, s2, s3, s4, s5,
             p0, p1):
        gsems = [g0, g1, g2, g3, g4, g5]
        ssems = [s0, s1, s2, s3, s4, s5]
        wid = lax.axis_index("s") * NUM_CORES + lax.axis_index("c")
        chunk_base = wid * chunks_per_w
        stage_base = pl.multiple_of(chunk_base - (chunk_base % 8), 8)
        off = chunk_base - stage_base

        s_first = chunk_base // nblk
        pos_base = pl.multiple_of(
            jnp.minimum(s_first - (s_first % 8), S - 16), 8)

        idx_cp = pltpu.async_copy(
            ids_hbm.at[pl.ds(stage_base, stage_rows)], idx_v, p0)
        pos_cp = pltpu.async_copy(ptab_hbm.at[pl.ds(pos_base, 16)], pos_v, p1)
        idx_cp.wait()

        def start_gather(c, b):
            pltpu.async_copy(
                ttab_hbm.at[idx_v.at[off + c]], bufs.at[b], gsems[b])

        def do_chunk(c, b, wait_store, prefetch):
            b2 = (b + 2) % NBUF
            pltpu.make_async_copy(
                ttab_hbm.at[idx_v.at[off + c]], bufs.at[b], gsems[b]).wait()
            if wait_store:
                pltpu.make_async_copy(
                    bufs.at[b2], out_hbm.at[oidx_v.at[b2]],
                    ssems[b2]).wait()
            if prefetch:
                start_gather(c + 2, b2)

            r_glob = chunk_base + c
            s_pos = r_glob // nblk
            obase = (r_glob % nblk) * (BW * S) + s_pos

            for j in range(vregs_per_row):
                sl = pl.ds(j * LANES, LANES)
                oidx_v[b, sl] = iota_v[sl] + obase

            pos_regs = [
                pos_v[s_pos - pos_base, pl.ds(j * LANES, LANES)]
                for j in range(vregs_per_row)
            ]

            def add_row(r, carry):
                for j in range(vregs_per_row):
                    sl = pl.ds(j * LANES, LANES)
                    bufs[b, r, sl] = bufs[b, r, sl] + pos_regs[j]
                return carry

            lax.fori_loop(0, BW, add_row, 0, unroll=4)

            pltpu.async_copy(
                bufs.at[b], out_hbm.at[oidx_v.at[b]], ssems[b])

        start_gather(0, 0)
        start_gather(1, 1)

        for j in range(vregs_per_row):
            iota_v[pl.ds(j * LANES, LANES)] = (
                lax.iota(jnp.int32, LANES) + j * LANES) * S
        pos_cp.wait()

        for b in range(NBUF):
            do_chunk(b, b, wait_store=(b >= 4), prefetch=True)

        def outer(cc, carry):
            for b in range(NBUF):
                do_chunk(cc * NBUF + b, b, wait_store=True, prefetch=True)
            return carry

        lax.fori_loop(1, n_full - 1, outer, 0)

        cl = (n_full - 1) * NBUF
        for b in range(NBUF):
            c = cl + b
            do_chunk(c, b, wait_store=True,
                     prefetch=(c + 2 < chunks_per_w))
        for t in range(n_tail):
            c = n_full * NBUF + t
            do_chunk(c, t, wait_store=True, prefetch=False)

        for c in range(chunks_per_w - 4, chunks_per_w):
            b = c % NBUF
            pltpu.make_async_copy(
                bufs.at[b], out_hbm.at[oidx_v.at[b]], ssems[b]).wait()

    return body(ids_t, token_table, pos_s)


def kernel(input_ids, token_table, position_table):
    B, S = input_ids.shape
    D = token_table.shape[1]
    ids_t = input_ids.T.reshape(-1, BW).astype(jnp.int32)
    pos_s = position_table[:S]
    out = _embed(ids_t, token_table, pos_s, S, D)
    return out.reshape(B, S, D)


# speedup vs baseline: 1.0073x; 1.0073x over previous
import functools

import jax
import jax.numpy as jnp
from jax import lax
from jax.experimental import pallas as pl
from jax.experimental.pallas import tpu as pltpu
from jax.experimental.pallas import tpu_sc as plsc

NUM_CORES = 2
NUM_SUBCORES = 16
NUM_WORKERS = NUM_CORES * NUM_SUBCORES
LANES = 16
BW = 128
NBUF = 5


@functools.partial(jax.jit, static_argnums=(3, 4))
def _embed(ids_t, token_table, pos_s, S, D):
    n_chunks = ids_t.shape[0]
    nblk = n_chunks // S
    n_rows = n_chunks * BW
    chunks_per_w = n_chunks // NUM_WORKERS
    n_outer = chunks_per_w // NBUF
    vregs_per_row = D // LANES
    stage_rows = chunks_per_w + 6

    mesh = plsc.VectorSubcoreMesh(
        core_axis_name="c", subcore_axis_name="s")

    @functools.partial(
        pl.kernel,
        mesh=mesh,
        out_type=jax.ShapeDtypeStruct((n_rows, D), jnp.float32),
        scratch_types=[
            pltpu.VMEM((stage_rows, BW), jnp.int32),
            pltpu.VMEM((16, D), jnp.float32),
            pltpu.VMEM((NBUF, BW, D), jnp.float32),
            pltpu.VMEM((NBUF, BW), jnp.int32),
            pltpu.VMEM((BW,), jnp.int32),
            pltpu.SemaphoreType.DMA,
            pltpu.SemaphoreType.DMA,
            pltpu.SemaphoreType.DMA,
            pltpu.SemaphoreType.DMA,
            pltpu.SemaphoreType.DMA,
            pltpu.SemaphoreType.DMA,
            pltpu.SemaphoreType.DMA,
            pltpu.SemaphoreType.DMA,
            pltpu.SemaphoreType.DMA,
            pltpu.SemaphoreType.DMA,
            pltpu.SemaphoreType.DMA,
            pltpu.SemaphoreType.DMA,
        ],
    )
    def body(ids_hbm, ttab_hbm, ptab_hbm, out_hbm, idx_v, pos_v, bufs,
             oidx_v, iota_v, g0, g1, g2, g3, g4, s0, s1, s2, s3, s4,
             p0, p1):
        gsems = [g0, g1, g2, g3, g4]
        ssems = [s0, s1, s2, s3, s4]
        wid = lax.axis_index("s") * NUM_CORES + lax.axis_index("c")
        chunk_base = wid * chunks_per_w
        stage_base = pl.multiple_of(chunk_base - (chunk_base % 8), 8)
        off = chunk_base - stage_base

        s_first = chunk_base // nblk
        pos_base = pl.multiple_of(
            jnp.minimum(s_first - (s_first % 8), S - 16), 8)

        idx_cp = pltpu.async_copy(
            ids_hbm.at[pl.ds(stage_base, stage_rows)], idx_v, p0)
        pos_cp = pltpu.async_copy(ptab_hbm.at[pl.ds(pos_base, 16)], pos_v, p1)
        idx_cp.wait()

        def start_gather(c, b):
            pltpu.async_copy(
                ttab_hbm.at[idx_v.at[off + c]], bufs.at[b], gsems[b])

        def do_chunk(c, b, wait_store, prefetch):
            b2 = (b + 2) % NBUF
            pltpu.make_async_copy(
                ttab_hbm.at[idx_v.at[off + c]], bufs.at[b], gsems[b]).wait()
            if wait_store:
                pltpu.make_async_copy(
                    bufs.at[b2], out_hbm.at[oidx_v.at[b2]],
                    ssems[b2]).wait()
            if prefetch:
                start_gather(c + 2, b2)

            r_glob = chunk_base + c
            s_pos = r_glob // nblk
            obase = (r_glob % nblk) * (BW * S) + s_pos

            for j in range(vregs_per_row):
                sl = pl.ds(j * LANES, LANES)
                oidx_v[b, sl] = iota_v[sl] + obase

            pos_regs = [
                pos_v[s_pos - pos_base, pl.ds(j * LANES, LANES)]
                for j in range(vregs_per_row)
            ]

            def add_row(r, carry):
                for j in range(vregs_per_row):
                    sl = pl.ds(j * LANES, LANES)
                    bufs[b, r, sl] = bufs[b, r, sl] + pos_regs[j]
                return carry

            lax.fori_loop(0, BW, add_row, 0, unroll=4)

            pltpu.async_copy(
                bufs.at[b], out_hbm.at[oidx_v.at[b]], ssems[b])

        start_gather(0, 0)
        start_gather(1, 1)

        for j in range(vregs_per_row):
            iota_v[pl.ds(j * LANES, LANES)] = (
                lax.iota(jnp.int32, LANES) + j * LANES) * S
        pos_cp.wait()

        for b in range(NBUF):
            do_chunk(b, b, wait_store=(b >= 3), prefetch=True)

        def outer(cc, carry):
            for b in range(NBUF):
                do_chunk(cc * NBUF + b, b, wait_store=True, prefetch=True)
            return carry

        lax.fori_loop(1, n_outer - 1, outer, 0)

        cl = (n_outer - 1) * NBUF
        for b in range(NBUF):
            do_chunk(cl + b, b, wait_store=True, prefetch=(b < 3))

        for b in (2, 3, 4):
            pltpu.make_async_copy(
                bufs.at[b], out_hbm.at[oidx_v.at[b]], ssems[b]).wait()

    return body(ids_t, token_table, pos_s)


def kernel(input_ids, token_table, position_table):
    B, S = input_ids.shape
    D = token_table.shape[1]
    ids_t = input_ids.T.reshape(-1, BW).astype(jnp.int32)
    pos_s = position_table[:S]
    out = _embed(ids_t, token_table, pos_s, S, D)
    return out.reshape(B, S, D)
